# Initial kernel scaffold; baseline (speedup 1.0000x reference)
import jax
import jax.numpy as jnp
from jax.experimental import pallas as pl


def kernel(states, edge_ids, training, type_W, type_b, gru_W, gru_U, gru_b):
    raise NotImplementedError("write your pallas kernel here")


# speedup vs baseline: 8.0429x; 8.0429x over previous
import functools

import jax
import jax.numpy as jnp
from jax import lax
from jax.experimental import pallas as pl
from jax.experimental.pallas import tpu as pltpu
from jax.experimental.pallas import tpu_sc as plsc

_N = 10000
_E = 320000
_D = 128
_T = 4
_STEPS = 4

_NPAD = 10240
_BN = 256
_CHUNK = 128
_NW = 32
_PW = 10112
_NCH = _PW // _CHUNK
_EPAD = _NW * _PW
_RPS = _NPAD // 16
_DUMMY = _N


def _types_kernel(h_ref, w_ref, b_ref, u_ref, gb_ref, y_ref, mh_ref):
    h = h_ref[...]
    for t in range(_T):
        y_ref[t] = (jnp.dot(h, w_ref[t], preferred_element_type=jnp.float32)
                    + b_ref[t][None, :])
    mh_ref[...] = (jnp.dot(h, u_ref[...], preferred_element_type=jnp.float32)
                   + gb_ref[1][None, :])


def _gru_kernel(p_ref, h_ref, mh_ref, w_ref, gb_ref, o_ref):
    msgs = p_ref[0] + p_ref[1]
    mx = (jnp.dot(msgs, w_ref[...], preferred_element_type=jnp.float32)
          + gb_ref[0][None, :])
    mh = mh_ref[...]
    h = h_ref[...]
    z = jax.nn.sigmoid(mx[:, :_D] + mh[:, :_D])
    r = jax.nn.sigmoid(mx[:, _D:2 * _D] + mh[:, _D:2 * _D])
    hh = jnp.tanh(mx[:, 2 * _D:] + r * mh[:, 2 * _D:])
    o_ref[...] = z * h + (1.0 - z) * hh


_sc_mesh = plsc.VectorSubcoreMesh(core_axis_name="c", subcore_axis_name="s")


@functools.partial(
    pl.kernel,
    mesh=_sc_mesh,
    out_type=jax.ShapeDtypeStruct((2, _NPAD, _D), jnp.float32),
    scratch_types=[
        pltpu.VMEM((_CHUNK,), jnp.int32),
        pltpu.VMEM((_CHUNK,), jnp.int32),
        pltpu.VMEM((_CHUNK, _D), jnp.float32),
        pltpu.VMEM_SHARED((_NPAD, _D), jnp.float32),
        pltpu.SemaphoreType.DMA,
    ],
)
def _edge_pass(y_hbm, gidx_hbm, didx_hbm, z_hbm, out_hbm,
               gi_v, di_v, rows_v, acc_sh, sem):
    c = lax.axis_index("c")
    s = lax.axis_index("s")
    wid = s * 2 + c
    pltpu.sync_copy(z_hbm.at[pl.ds(s * _RPS, _RPS)],
                    acc_sh.at[pl.ds(s * _RPS, _RPS)])
    plsc.subcore_barrier()

    def body(j, carry):
        base = wid * _PW + j * _CHUNK
        pltpu.sync_copy(gidx_hbm.at[pl.ds(base, _CHUNK)], gi_v)
        pltpu.async_copy(y_hbm.at[gi_v], rows_v, sem).wait()
        pltpu.sync_copy(didx_hbm.at[pl.ds(base, _CHUNK)], di_v)
        pltpu.sync_copy(rows_v, acc_sh.at[di_v], add=True)
        return carry

    lax.fori_loop(0, _NCH, body, 0)
    plsc.subcore_barrier()
    pltpu.sync_copy(acc_sh.at[pl.ds(s * _RPS, _RPS)],
                    out_hbm.at[c, pl.ds(s * _RPS, _RPS)])


def kernel(states, edge_ids, training, type_W, type_b, gru_W, gru_U, gru_b):
    etype = edge_ids[:, 0]
    src = edge_ids[:, 1]
    dst = edge_ids[:, 2]
    pad = _EPAD - _E
    gidx = jnp.concatenate([etype * _NPAD + src,
                            jnp.zeros((pad,), jnp.int32)])
    didx = jnp.concatenate([dst, jnp.full((pad,), _DUMMY, jnp.int32)])
    h = jnp.zeros((_NPAD, _D), jnp.float32).at[:_N].set(states)
    zeros_nd = jnp.zeros((_NPAD, _D), jnp.float32)

    grid = _NPAD // _BN
    types_call = pl.pallas_call(
        _types_kernel,
        grid=(grid,),
        in_specs=[
            pl.BlockSpec((_BN, _D), lambda i: (i, 0)),
            pl.BlockSpec((_T, _D, _D), lambda i: (0, 0, 0)),
            pl.BlockSpec((_T, _D), lambda i: (0, 0)),
            pl.BlockSpec((_D, 3 * _D), lambda i: (0, 0)),
            pl.BlockSpec((2, 3 * _D), lambda i: (0, 0)),
        ],
        out_specs=[
            pl.BlockSpec((_T, _BN, _D), lambda i: (0, i, 0)),
            pl.BlockSpec((_BN, 3 * _D), lambda i: (i, 0)),
        ],
        out_shape=[
            jax.ShapeDtypeStruct((_T, _NPAD, _D), jnp.float32),
            jax.ShapeDtypeStruct((_NPAD, 3 * _D), jnp.float32),
        ],
    )
    gru_call = pl.pallas_call(
        _gru_kernel,
        grid=(grid,),
        in_specs=[
            pl.BlockSpec((2, _BN, _D), lambda i: (0, i, 0)),
            pl.BlockSpec((_BN, _D), lambda i: (i, 0)),
            pl.BlockSpec((_BN, 3 * _D), lambda i: (i, 0)),
            pl.BlockSpec((_D, 3 * _D), lambda i: (0, 0)),
            pl.BlockSpec((2, 3 * _D), lambda i: (0, 0)),
        ],
        out_specs=pl.BlockSpec((_BN, _D), lambda i: (i, 0)),
        out_shape=jax.ShapeDtypeStruct((_NPAD, _D), jnp.float32),
    )

    for _ in range(_STEPS):
        y, mh = types_call(h, type_W, type_b, gru_U, gru_b)
        parts = _edge_pass(y.reshape(_T * _NPAD, _D), gidx, didx, zeros_nd)
        h = gru_call(parts, h, mh, gru_W, gru_b)
    return h[:_N]
